# Initial kernel scaffold; baseline (speedup 1.0000x reference)
import jax
import jax.numpy as jnp
from jax.experimental import pallas as pl


def kernel(node_features, edge_features, edge_index, W1, b1, g1, be1, W2, b2, g2, be2):
    raise NotImplementedError("write your pallas kernel here")


# speedup vs baseline: 3.6085x; 3.6085x over previous
import functools

import jax
import jax.numpy as jnp
from jax import lax
from jax.experimental import pallas as pl
from jax.experimental.pallas import tpu as pltpu
from jax.experimental.pallas import tpu_sc as plsc

N = 10000
E = 320000
DN = 128
DE = 16

NC = 2
NS = 16
NW = NC * NS
EPW = E // NW
SUB = 80
CH = 400
NSUB = CH // SUB
NCH = EPW // CH
SPW = EPW // SUB
NP = 10240
RPS = NP // NS


def _proj_body(nf_ref, wn_ref, ps_ref, pt_ref):
    p = jnp.dot(nf_ref[...], wn_ref[...], preferred_element_type=jnp.float32)
    ps_ref[...] = p[:, :DE]
    pt_ref[...] = p[:, DE:]


def _proj(node_features, wn):
    bn = 1000
    return pl.pallas_call(
        _proj_body,
        grid=(N // bn,),
        in_specs=[
            pl.BlockSpec((bn, DN), lambda i: (i, 0)),
            pl.BlockSpec((DN, 2 * DE), lambda i: (0, 0)),
        ],
        out_specs=[
            pl.BlockSpec((bn, DE), lambda i: (i, 0)),
            pl.BlockSpec((bn, DE), lambda i: (i, 0)),
        ],
        out_shape=[
            jax.ShapeDtypeStruct((N, DE), jnp.float32),
            jax.ShapeDtypeStruct((N, DE), jnp.float32),
        ],
    )(node_features, wn)


def _elin_body(ef_ref, w_ref, b_ref, out_ref):
    x = ef_ref[...]
    out_ref[...] = x + jnp.dot(x, w_ref[...], preferred_element_type=jnp.float32) + b_ref[...]


def _elin(edge_features, w1a, b1row):
    be = 3200
    return pl.pallas_call(
        _elin_body,
        grid=(E // be,),
        in_specs=[
            pl.BlockSpec((be, DE), lambda i: (i, 0)),
            pl.BlockSpec((DE, DE), lambda i: (0, 0)),
            pl.BlockSpec((1, DE), lambda i: (0, 0)),
        ],
        out_specs=pl.BlockSpec((be, DE), lambda i: (i, 0)),
        out_shape=jax.ShapeDtypeStruct((E, DE), jnp.float32),
    )(edge_features, w1a, b1row)


_GATHER_DNUMS = lax.GatherDimensionNumbers(
    offset_dims=(), collapsed_slice_dims=(0,), start_index_map=(0,))


def _perm16(x, idx):
    return lax.gather(x, idx[:, None], _GATHER_DNUMS, slice_sizes=(1,),
                      mode=lax.GatherScatterMode.PROMISE_IN_BOUNDS)


def _vsum16(x):
    ids = lax.iota(jnp.int32, 16)
    for sh in (8, 4, 2, 1):
        x = x + _perm16(x, ids ^ sh)
    return x


def _fast_rsqrt16(v):
    i = lax.bitcast_convert_type(v, jnp.int32)
    i = 0x5F3759DF - (i >> 1)
    y = lax.bitcast_convert_type(i, jnp.float32)
    y = y * (1.5 - 0.5 * v * y * y)
    y = y * (1.5 - 0.5 * v * y * y)
    y = y * (1.5 - 0.5 * v * y * y)
    return y


_sc_mesh = plsc.VectorSubcoreMesh(
    core_axis_name="c", subcore_axis_name="s", num_cores=NC, num_subcores=NS
)


@functools.partial(
    pl.kernel,
    out_type=(
        jax.ShapeDtypeStruct((E, DE), jnp.float32),
        jax.ShapeDtypeStruct((NC, NP, 2 * DE), jnp.float32),
    ),
    mesh=_sc_mesh,
    scratch_types=[
        pltpu.VMEM((NSUB, SUB), jnp.int32),
        pltpu.VMEM((NSUB, SUB), jnp.int32),
        pltpu.VMEM((CH, DE), jnp.float32),
        pltpu.VMEM((CH, DE), jnp.float32),
        pltpu.VMEM((CH, DE), jnp.float32),
        pltpu.VMEM((CH, DE), jnp.float32),
        pltpu.VMEM((CH, 2 * DE), jnp.float32),
        pltpu.VMEM((RPS, 2 * DE), jnp.float32),
        pltpu.VMEM((DE,), jnp.float32),
        pltpu.VMEM((DE,), jnp.float32),
        pltpu.VMEM_SHARED((NP, 2 * DE), jnp.float32),
        pltpu.SemaphoreType.DMA,
    ],
    compiler_params=pltpu.CompilerParams(use_tc_tiling_on_sc=False),
)
def _sc_edge(elin_hbm, src_hbm, tgt_hbm, psrc_hbm, ptgt_hbm, g1_hbm, be1_hbm,
             newe_hbm, part_hbm,
             sidx_v, tidx_v, elin_v, gsrc_v, gtgt_v, newe_v, vals_v, zbuf_v,
             g1_v, be1_v, acc_sh, sem):
    c = lax.axis_index("c")
    s = lax.axis_index("s")
    w = s * NC + c

    z16 = jnp.zeros((DE,), jnp.float32)
    o16 = jnp.full((DE,), 1.0, jnp.float32)

    def zb_body(i, carry):
        zbuf_v[i, pl.ds(0, DE)] = z16
        zbuf_v[i, pl.ds(DE, DE)] = z16
        return carry

    lax.fori_loop(0, RPS, zb_body, 0)
    pltpu.sync_copy(zbuf_v, acc_sh.at[pl.ds(s * RPS, RPS)])

    def ones_body(i, carry):
        vals_v[i, pl.ds(DE, DE)] = o16
        return carry

    lax.fori_loop(0, CH, ones_body, 0)

    pltpu.sync_copy(g1_hbm, g1_v)
    pltpu.sync_copy(be1_hbm, be1_v)
    g1 = g1_v[...]
    be1 = be1_v[...]

    plsc.subcore_barrier()

    def chunk_body(k, carry):
        base = w * EPW + k * CH
        irow = w * SPW + k * NSUB
        pltpu.sync_copy(src_hbm.at[pl.ds(irow, NSUB)], sidx_v)
        pltpu.sync_copy(tgt_hbm.at[pl.ds(irow, NSUB)], tidx_v)
        h_el = pltpu.async_copy(elin_hbm.at[pl.ds(base, CH)], elin_v, sem)
        hs = []
        for j in range(NSUB):
            hs.append(pltpu.async_copy(
                psrc_hbm.at[sidx_v.at[j]],
                gsrc_v.at[pl.ds(j * SUB, SUB)], sem))
            hs.append(pltpu.async_copy(
                ptgt_hbm.at[tidx_v.at[j]],
                gtgt_v.at[pl.ds(j * SUB, SUB)], sem))
        h_el.wait()
        for h in hs:
            h.wait()

        def edge_body(i, carry2):
            pre = elin_v[i, :] + gsrc_v[i, :] + gtgt_v[i, :]
            mu = _vsum16(pre) * (1.0 / DE)
            d = pre - mu
            vv = _vsum16(d * d) * (1.0 / DE) + 1e-5
            out = d * _fast_rsqrt16(vv) * g1 + be1
            newe_v[i, :] = out
            vals_v[i, pl.ds(0, DE)] = out
            return carry2

        lax.fori_loop(0, CH, edge_body, 0)

        for j in range(NSUB):
            vrow = vals_v.at[pl.ds(j * SUB, SUB)]
            pltpu.sync_copy(vrow, acc_sh.at[sidx_v.at[j]], add=True)
            pltpu.sync_copy(vrow, acc_sh.at[tidx_v.at[j]], add=True)
        pltpu.sync_copy(newe_v, newe_hbm.at[pl.ds(base, CH)])
        return carry

    lax.fori_loop(0, NCH, chunk_body, 0)

    plsc.subcore_barrier()
    pltpu.sync_copy(acc_sh.at[pl.ds(s * RPS, RPS)],
                    part_hbm.at[c, pl.ds(s * RPS, RPS)])


def _final_body(nf_ref, p0_ref, p1_ref, w2a_ref, w2b_ref, b2_ref, g2_ref,
                be2_ref, out_ref):
    sums = p0_ref[...] + p1_ref[...]
    emean = sums[:, :DE] / (sums[:, DE:DE + 1] + 1e-10)
    nf = nf_ref[...]
    h = (nf
         + jnp.dot(nf, w2a_ref[...], preferred_element_type=jnp.float32)
         + jnp.dot(emean, w2b_ref[...], preferred_element_type=jnp.float32)
         + b2_ref[...])
    mu = jnp.mean(h, axis=-1, keepdims=True)
    d = h - mu
    var = jnp.mean(d * d, axis=-1, keepdims=True)
    out_ref[...] = d * lax.rsqrt(var + 1e-5) * g2_ref[...] + be2_ref[...]


def _final(node_features, p0, p1, w2a, w2b, b2row, g2row, be2row):
    bn = 400
    return pl.pallas_call(
        _final_body,
        grid=(N // bn,),
        in_specs=[
            pl.BlockSpec((bn, DN), lambda i: (i, 0)),
            pl.BlockSpec((bn, 2 * DE), lambda i: (i, 0)),
            pl.BlockSpec((bn, 2 * DE), lambda i: (i, 0)),
            pl.BlockSpec((DN, DN), lambda i: (0, 0)),
            pl.BlockSpec((DE, DN), lambda i: (0, 0)),
            pl.BlockSpec((1, DN), lambda i: (0, 0)),
            pl.BlockSpec((1, DN), lambda i: (0, 0)),
            pl.BlockSpec((1, DN), lambda i: (0, 0)),
        ],
        out_specs=pl.BlockSpec((bn, DN), lambda i: (i, 0)),
        out_shape=jax.ShapeDtypeStruct((N, DN), jnp.float32),
    )(node_features, p0, p1, w2a, w2b, b2row, g2row, be2row)


def kernel(node_features, edge_features, edge_index, W1, b1, g1, be1,
           W2, b2, g2, be2):
    src3 = edge_index[0].reshape(E // SUB, SUB)
    tgt3 = edge_index[1].reshape(E // SUB, SUB)

    wn = jnp.concatenate([W1[DE:DE + DN], W1[DE + DN:]], axis=1)
    psrc, ptgt = _proj(node_features, wn)
    elin = _elin(edge_features, W1[:DE], b1.reshape(1, DE))

    new_edges, partials = _sc_edge(elin, src3, tgt3, psrc, ptgt, g1, be1)

    new_nodes = _final(node_features, partials[0, :N], partials[1, :N],
                       W2[:DN], W2[DN:], b2.reshape(1, DN),
                       g2.reshape(1, DN), be2.reshape(1, DN))
    return (new_nodes, new_edges)


# speedup vs baseline: 9.1704x; 2.5414x over previous
import functools

import jax
import jax.numpy as jnp
from jax import lax
from jax.experimental import pallas as pl
from jax.experimental.pallas import tpu as pltpu
from jax.experimental.pallas import tpu_sc as plsc

N = 10000
E = 320000
DN = 128
DE = 16
EP = E // 8

NC = 2
NS = 16
NW = NC * NS
EPW = E // NW
SUB = 100
CH = 1000
NSUB = CH // SUB
NCH = EPW // CH
SPW = EPW // SUB
NP = 10240
RPS = NP // NS


def _proj_body(nf_ref, wn_ref, ps_ref, pt_ref):
    p = jnp.dot(nf_ref[...], wn_ref[...], preferred_element_type=jnp.float32)
    ps_ref[...] = p[:, :DE]
    pt_ref[...] = p[:, DE:]


def _proj(node_features, wn):
    bn = 1000
    return pl.pallas_call(
        _proj_body,
        grid=(N // bn,),
        in_specs=[
            pl.BlockSpec((bn, DN), lambda i: (i, 0)),
            pl.BlockSpec((DN, 2 * DE), lambda i: (0, 0)),
        ],
        out_specs=[
            pl.BlockSpec((bn, DE), lambda i: (i, 0)),
            pl.BlockSpec((bn, DE), lambda i: (i, 0)),
        ],
        out_shape=[
            jax.ShapeDtypeStruct((N, DE), jnp.float32),
            jax.ShapeDtypeStruct((N, DE), jnp.float32),
        ],
    )(node_features, wn)


_sc_mesh = plsc.VectorSubcoreMesh(
    core_axis_name="c", subcore_axis_name="s", num_cores=NC, num_subcores=NS
)


@functools.partial(
    pl.kernel,
    out_type=(
        jax.ShapeDtypeStruct((E, DE), jnp.float32),
        jax.ShapeDtypeStruct((E, DE), jnp.float32),
    ),
    mesh=_sc_mesh,
    scratch_types=[
        pltpu.VMEM((NSUB, SUB), jnp.int32),
        pltpu.VMEM((NSUB, SUB), jnp.int32),
        pltpu.VMEM((CH, DE), jnp.float32),
        pltpu.VMEM((CH, DE), jnp.float32),
        pltpu.SemaphoreType.DMA,
    ],
    compiler_params=pltpu.CompilerParams(use_tc_tiling_on_sc=False),
)
def _sc_gather(src_hbm, tgt_hbm, psrc_hbm, ptgt_hbm,
               gsrc_hbm, gtgt_hbm,
               sidx_v, tidx_v, gsrc_v, gtgt_v, sem):
    c = lax.axis_index("c")
    s = lax.axis_index("s")
    w = s * NC + c

    def chunk_body(k, carry):
        base = w * EPW + k * CH
        irow = w * SPW + k * NSUB
        pltpu.sync_copy(src_hbm.at[pl.ds(irow, NSUB)], sidx_v)
        pltpu.sync_copy(tgt_hbm.at[pl.ds(irow, NSUB)], tidx_v)
        hs = []
        for j in range(NSUB):
            hs.append(pltpu.async_copy(
                psrc_hbm.at[sidx_v.at[j]],
                gsrc_v.at[pl.ds(j * SUB, SUB)], sem))
            hs.append(pltpu.async_copy(
                ptgt_hbm.at[tidx_v.at[j]],
                gtgt_v.at[pl.ds(j * SUB, SUB)], sem))
        for h in hs:
            h.wait()
        pltpu.sync_copy(gsrc_v, gsrc_hbm.at[pl.ds(base, CH)])
        pltpu.sync_copy(gtgt_v, gtgt_hbm.at[pl.ds(base, CH)])
        return carry

    lax.fori_loop(0, NCH, chunk_body, 0)


def _eln_body(x_ref, gs_ref, gt_ref, a_ref, m_ref, b1_ref, g1_ref, be1_ref,
              out_ref):
    x = x_ref[...]
    y = (lax.dot(x, a_ref[...], precision=lax.Precision.HIGHEST,
                 preferred_element_type=jnp.float32)
         + b1_ref[...] + gs_ref[...] + gt_ref[...])
    mu = lax.dot(y, m_ref[...], precision=lax.Precision.HIGHEST,
                 preferred_element_type=jnp.float32)
    d = y - mu
    var = lax.dot(d * d, m_ref[...], precision=lax.Precision.HIGHEST,
                  preferred_element_type=jnp.float32)
    out_ref[...] = d * lax.rsqrt(var + 1e-5) * g1_ref[...] + be1_ref[...]


def _eln(xp, gsp, gtp, a128, m128, b1t, g1t, be1t):
    bp = 2000
    return pl.pallas_call(
        _eln_body,
        grid=(EP // bp,),
        in_specs=[
            pl.BlockSpec((bp, DN), lambda i: (i, 0)),
            pl.BlockSpec((bp, DN), lambda i: (i, 0)),
            pl.BlockSpec((bp, DN), lambda i: (i, 0)),
            pl.BlockSpec((DN, DN), lambda i: (0, 0)),
            pl.BlockSpec((DN, DN), lambda i: (0, 0)),
            pl.BlockSpec((1, DN), lambda i: (0, 0)),
            pl.BlockSpec((1, DN), lambda i: (0, 0)),
            pl.BlockSpec((1, DN), lambda i: (0, 0)),
        ],
        out_specs=pl.BlockSpec((bp, DN), lambda i: (i, 0)),
        out_shape=jax.ShapeDtypeStruct((EP, DN), jnp.float32),
    )(xp, gsp, gtp, a128, m128, b1t, g1t, be1t)


@functools.partial(
    pl.kernel,
    out_type=jax.ShapeDtypeStruct((NC, NP, 2 * DE), jnp.float32),
    mesh=_sc_mesh,
    scratch_types=[
        pltpu.VMEM((NSUB, SUB), jnp.int32),
        pltpu.VMEM((NSUB, SUB), jnp.int32),
        pltpu.VMEM((CH, 2 * DE), jnp.float32),
        pltpu.VMEM((RPS, 2 * DE), jnp.float32),
        pltpu.VMEM_SHARED((NP, 2 * DE), jnp.float32),
        pltpu.SemaphoreType.DMA,
    ],
    compiler_params=pltpu.CompilerParams(use_tc_tiling_on_sc=False),
)
def _sc_scatter(newe_hbm, src_hbm, tgt_hbm, part_hbm,
                sidx_v, tidx_v, vals_v, zbuf_v, acc_sh, sem):
    c = lax.axis_index("c")
    s = lax.axis_index("s")
    w = s * NC + c

    z16 = jnp.zeros((DE,), jnp.float32)
    o16 = jnp.full((DE,), 1.0, jnp.float32)

    def zb_body(i, carry):
        zbuf_v[i, pl.ds(0, DE)] = z16
        zbuf_v[i, pl.ds(DE, DE)] = z16
        return carry

    lax.fori_loop(0, RPS, zb_body, 0)
    pltpu.sync_copy(zbuf_v, acc_sh.at[pl.ds(s * RPS, RPS)])

    def ones_body(i, carry):
        vals_v[i, pl.ds(DE, DE)] = o16
        return carry

    lax.fori_loop(0, CH, ones_body, 0)

    plsc.subcore_barrier()

    def chunk_body(k, carry):
        base = w * EPW + k * CH
        irow = w * SPW + k * NSUB
        pltpu.sync_copy(src_hbm.at[pl.ds(irow, NSUB)], sidx_v)
        pltpu.sync_copy(tgt_hbm.at[pl.ds(irow, NSUB)], tidx_v)
        pltpu.sync_copy(newe_hbm.at[pl.ds(base, CH)],
                        vals_v.at[:, pl.ds(0, DE)])
        for j in range(NSUB):
            vrow = vals_v.at[pl.ds(j * SUB, SUB)]
            pltpu.sync_copy(vrow, acc_sh.at[sidx_v.at[j]], add=True)
            pltpu.sync_copy(vrow, acc_sh.at[tidx_v.at[j]], add=True)
        return carry

    lax.fori_loop(0, NCH, chunk_body, 0)

    plsc.subcore_barrier()
    pltpu.sync_copy(acc_sh.at[pl.ds(s * RPS, RPS)],
                    part_hbm.at[c, pl.ds(s * RPS, RPS)])


def _final_body(nf_ref, p0_ref, p1_ref, w2a_ref, w2b_ref, b2_ref, g2_ref,
                be2_ref, out_ref):
    sums = p0_ref[...] + p1_ref[...]
    emean = sums[:, :DE] / (sums[:, DE:DE + 1] + 1e-10)
    nf = nf_ref[...]
    h = (nf
         + jnp.dot(nf, w2a_ref[...], preferred_element_type=jnp.float32)
         + jnp.dot(emean, w2b_ref[...], preferred_element_type=jnp.float32)
         + b2_ref[...])
    mu = jnp.mean(h, axis=-1, keepdims=True)
    d = h - mu
    var = jnp.mean(d * d, axis=-1, keepdims=True)
    out_ref[...] = d * lax.rsqrt(var + 1e-5) * g2_ref[...] + be2_ref[...]


def _final(node_features, p0, p1, w2a, w2b, b2row, g2row, be2row):
    bn = 400
    return pl.pallas_call(
        _final_body,
        grid=(N // bn,),
        in_specs=[
            pl.BlockSpec((bn, DN), lambda i: (i, 0)),
            pl.BlockSpec((bn, 2 * DE), lambda i: (i, 0)),
            pl.BlockSpec((bn, 2 * DE), lambda i: (i, 0)),
            pl.BlockSpec((DN, DN), lambda i: (0, 0)),
            pl.BlockSpec((DE, DN), lambda i: (0, 0)),
            pl.BlockSpec((1, DN), lambda i: (0, 0)),
            pl.BlockSpec((1, DN), lambda i: (0, 0)),
            pl.BlockSpec((1, DN), lambda i: (0, 0)),
        ],
        out_specs=pl.BlockSpec((bn, DN), lambda i: (i, 0)),
        out_shape=jax.ShapeDtypeStruct((N, DN), jnp.float32),
    )(node_features, p0, p1, w2a, w2b, b2row, g2row, be2row)


def kernel(node_features, edge_features, edge_index, W1, b1, g1, be1,
           W2, b2, g2, be2):
    src3 = edge_index[0].reshape(E // SUB, SUB)
    tgt3 = edge_index[1].reshape(E // SUB, SUB)

    wn = jnp.concatenate([W1[DE:DE + DN], W1[DE + DN:]], axis=1)
    psrc, ptgt = _proj(node_features, wn)

    gsrc, gtgt = _sc_gather(src3, tgt3, psrc, ptgt)

    eye8 = jnp.eye(8, dtype=jnp.float32)
    a128 = jnp.eye(DN, dtype=jnp.float32) + jnp.kron(eye8, W1[:DE])
    m128 = jnp.kron(eye8, jnp.full((DE, DE), 1.0 / DE, jnp.float32))
    b1t = jnp.tile(b1, 8).reshape(1, DN)
    g1t = jnp.tile(g1, 8).reshape(1, DN)
    be1t = jnp.tile(be1, 8).reshape(1, DN)

    newe_p = _eln(edge_features.reshape(EP, DN),
                  gsrc.reshape(EP, DN), gtgt.reshape(EP, DN),
                  a128, m128, b1t, g1t, be1t)
    new_edges = newe_p.reshape(E, DE)

    partials = _sc_scatter(new_edges, src3, tgt3)

    new_nodes = _final(node_features, partials[0, :N], partials[1, :N],
                       W2[:DN], W2[DN:], b2.reshape(1, DN),
                       g2.reshape(1, DN), be2.reshape(1, DN))
    return (new_nodes, new_edges)


# speedup vs baseline: 10.3631x; 1.1301x over previous
import functools

import jax
import jax.numpy as jnp
from jax import lax
from jax.experimental import pallas as pl
from jax.experimental.pallas import tpu as pltpu
from jax.experimental.pallas import tpu_sc as plsc

N = 10000
E = 320000
DN = 128
DE = 16
EP = E // 8

NC = 2
NS = 16
NW = NC * NS
EPW = E // NW
SUB = 100
CH = 1000
NSUB = CH // SUB
NCH = EPW // CH
SPW = EPW // SUB
NP = 10240
RPS = NP // NS
CROWS = CH * DE // DN


def _proj_body(nf_ref, wn_ref, ps_ref, pt_ref):
    p = jnp.dot(nf_ref[...], wn_ref[...], preferred_element_type=jnp.float32)
    ps_ref[...] = p[:, :DE]
    pt_ref[...] = p[:, DE:]


def _proj(node_features, wn):
    bn = 1000
    return pl.pallas_call(
        _proj_body,
        grid=(N // bn,),
        in_specs=[
            pl.BlockSpec((bn, DN), lambda i: (i, 0)),
            pl.BlockSpec((DN, 2 * DE), lambda i: (0, 0)),
        ],
        out_specs=[
            pl.BlockSpec((bn, DE), lambda i: (i, 0)),
            pl.BlockSpec((bn, DE), lambda i: (i, 0)),
        ],
        out_shape=[
            jax.ShapeDtypeStruct((N, DE), jnp.float32),
            jax.ShapeDtypeStruct((N, DE), jnp.float32),
        ],
    )(node_features, wn)


_sc_mesh = plsc.VectorSubcoreMesh(
    core_axis_name="c", subcore_axis_name="s", num_cores=NC, num_subcores=NS
)


@functools.partial(
    pl.kernel,
    out_type=(
        jax.ShapeDtypeStruct((E, DE), jnp.float32),
        jax.ShapeDtypeStruct((E, DE), jnp.float32),
    ),
    mesh=_sc_mesh,
    scratch_types=[
        pltpu.VMEM((NSUB, SUB), jnp.int32),
        pltpu.VMEM((NSUB, SUB), jnp.int32),
        pltpu.VMEM((CH, DE), jnp.float32),
        pltpu.VMEM((CH, DE), jnp.float32),
        pltpu.SemaphoreType.DMA,
    ],
    compiler_params=pltpu.CompilerParams(use_tc_tiling_on_sc=False),
)
def _sc_gather(src_hbm, tgt_hbm, psrc_hbm, ptgt_hbm,
               gsrc_hbm, gtgt_hbm,
               sidx_v, tidx_v, gsrc_v, gtgt_v, sem):
    c = lax.axis_index("c")
    s = lax.axis_index("s")
    w = s * NC + c

    def chunk_body(k, carry):
        base = w * EPW + k * CH
        irow = w * SPW + k * NSUB
        pltpu.sync_copy(src_hbm.at[pl.ds(irow, NSUB)], sidx_v)
        pltpu.sync_copy(tgt_hbm.at[pl.ds(irow, NSUB)], tidx_v)
        hs = []
        for j in range(NSUB):
            hs.append(pltpu.async_copy(
                psrc_hbm.at[sidx_v.at[j]],
                gsrc_v.at[pl.ds(j * SUB, SUB)], sem))
            hs.append(pltpu.async_copy(
                ptgt_hbm.at[tidx_v.at[j]],
                gtgt_v.at[pl.ds(j * SUB, SUB)], sem))
        for h in hs:
            h.wait()
        pltpu.sync_copy(gsrc_v, gsrc_hbm.at[pl.ds(base, CH)])
        pltpu.sync_copy(gtgt_v, gtgt_hbm.at[pl.ds(base, CH)])
        return carry

    lax.fori_loop(0, NCH, chunk_body, 0)


def _eln_body(x_ref, gs_ref, gt_ref, a_ref, m_ref, b1_ref, g1_ref, be1_ref,
              out_ref):
    x = x_ref[...]
    y = (lax.dot(x, a_ref[...], preferred_element_type=jnp.float32)
         + b1_ref[...] + gs_ref[...] + gt_ref[...])
    mu = lax.dot(y, m_ref[...], preferred_element_type=jnp.float32)
    d = y - mu
    var = lax.dot(d * d, m_ref[...], preferred_element_type=jnp.float32)
    out_ref[...] = d * lax.rsqrt(var + 1e-5) * g1_ref[...] + be1_ref[...]


def _eln(xp, gsp, gtp, a128, m128, b1t, g1t, be1t):
    bp = 2000
    return pl.pallas_call(
        _eln_body,
        grid=(EP // bp,),
        in_specs=[
            pl.BlockSpec((bp, DN), lambda i: (i, 0)),
            pl.BlockSpec((bp, DN), lambda i: (i, 0)),
            pl.BlockSpec((bp, DN), lambda i: (i, 0)),
            pl.BlockSpec((DN, DN), lambda i: (0, 0)),
            pl.BlockSpec((DN, DN), lambda i: (0, 0)),
            pl.BlockSpec((1, DN), lambda i: (0, 0)),
            pl.BlockSpec((1, DN), lambda i: (0, 0)),
            pl.BlockSpec((1, DN), lambda i: (0, 0)),
        ],
        out_specs=pl.BlockSpec((bp, DN), lambda i: (i, 0)),
        out_shape=jax.ShapeDtypeStruct((EP, DN), jnp.float32),
    )(xp, gsp, gtp, a128, m128, b1t, g1t, be1t)


@functools.partial(
    pl.kernel,
    out_type=jax.ShapeDtypeStruct((NC, NP, 2 * DE), jnp.float32),
    mesh=_sc_mesh,
    scratch_types=[
        pltpu.VMEM((NSUB, SUB), jnp.int32),
        pltpu.VMEM((NSUB, SUB), jnp.int32),
        pltpu.VMEM((CH, 2 * DE), jnp.float32),
        pltpu.VMEM((RPS, 2 * DE), jnp.float32),
        pltpu.VMEM_SHARED((NP, 2 * DE), jnp.float32),
        pltpu.SemaphoreType.DMA,
    ],
    compiler_params=pltpu.CompilerParams(use_tc_tiling_on_sc=False),
)
def _sc_scatter(newe_hbm, src_hbm, tgt_hbm, part_hbm,
                sidx_v, tidx_v, vals_v, zbuf_v, acc_sh, sem):
    c = lax.axis_index("c")
    s = lax.axis_index("s")
    w = s * NC + c

    z16 = jnp.zeros((DE,), jnp.float32)
    o16 = jnp.full((DE,), 1.0, jnp.float32)

    def zb_body(i, carry):
        zbuf_v[i, pl.ds(0, DE)] = z16
        zbuf_v[i, pl.ds(DE, DE)] = z16
        return carry

    lax.fori_loop(0, RPS, zb_body, 0)
    pltpu.sync_copy(zbuf_v, acc_sh.at[pl.ds(s * RPS, RPS)])

    def ones_body(i, carry):
        vals_v[i, pl.ds(DE, DE)] = o16
        return carry

    lax.fori_loop(0, CH, ones_body, 0)

    plsc.subcore_barrier()

    def chunk_body(k, carry):
        base = w * EPW + k * CH
        irow = w * SPW + k * NSUB
        pltpu.sync_copy(src_hbm.at[pl.ds(irow, NSUB)], sidx_v)
        pltpu.sync_copy(tgt_hbm.at[pl.ds(irow, NSUB)], tidx_v)
        pltpu.sync_copy(newe_hbm.at[pl.ds(base, CH)],
                        vals_v.at[:, pl.ds(0, DE)])
        for j in range(NSUB):
            vrow = vals_v.at[pl.ds(j * SUB, SUB)]
            pltpu.sync_copy(vrow, acc_sh.at[sidx_v.at[j]], add=True)
            pltpu.sync_copy(vrow, acc_sh.at[tidx_v.at[j]], add=True)
        return carry

    lax.fori_loop(0, NCH, chunk_body, 0)

    plsc.subcore_barrier()
    pltpu.sync_copy(acc_sh.at[pl.ds(s * RPS, RPS)],
                    part_hbm.at[c, pl.ds(s * RPS, RPS)])


def _final_body(nf_ref, p0_ref, p1_ref, w2a_ref, w2b_ref, b2_ref, g2_ref,
                be2_ref, out_ref):
    sums = p0_ref[...] + p1_ref[...]
    emean = sums[:, :DE] / (sums[:, DE:DE + 1] + 1e-10)
    nf = nf_ref[...]
    h = (nf
         + jnp.dot(nf, w2a_ref[...], preferred_element_type=jnp.float32)
         + jnp.dot(emean, w2b_ref[...], preferred_element_type=jnp.float32)
         + b2_ref[...])
    mu = jnp.mean(h, axis=-1, keepdims=True)
    d = h - mu
    var = jnp.mean(d * d, axis=-1, keepdims=True)
    out_ref[...] = d * lax.rsqrt(var + 1e-5) * g2_ref[...] + be2_ref[...]


def _final(node_features, p0, p1, w2a, w2b, b2row, g2row, be2row):
    bn = 400
    return pl.pallas_call(
        _final_body,
        grid=(N // bn,),
        in_specs=[
            pl.BlockSpec((bn, DN), lambda i: (i, 0)),
            pl.BlockSpec((bn, 2 * DE), lambda i: (i, 0)),
            pl.BlockSpec((bn, 2 * DE), lambda i: (i, 0)),
            pl.BlockSpec((DN, DN), lambda i: (0, 0)),
            pl.BlockSpec((DE, DN), lambda i: (0, 0)),
            pl.BlockSpec((1, DN), lambda i: (0, 0)),
            pl.BlockSpec((1, DN), lambda i: (0, 0)),
            pl.BlockSpec((1, DN), lambda i: (0, 0)),
        ],
        out_specs=pl.BlockSpec((bn, DN), lambda i: (i, 0)),
        out_shape=jax.ShapeDtypeStruct((N, DN), jnp.float32),
    )(node_features, p0, p1, w2a, w2b, b2row, g2row, be2row)


def kernel(node_features, edge_features, edge_index, W1, b1, g1, be1,
           W2, b2, g2, be2):
    src3 = edge_index[0].reshape(E // SUB, SUB)
    tgt3 = edge_index[1].reshape(E // SUB, SUB)

    wn = jnp.concatenate([W1[DE:DE + DN], W1[DE + DN:]], axis=1)
    psrc, ptgt = _proj(node_features, wn)

    gsrc, gtgt = _sc_gather(src3, tgt3, psrc, ptgt)

    eye8 = jnp.eye(8, dtype=jnp.float32)
    a128 = jnp.eye(DN, dtype=jnp.float32) + jnp.kron(eye8, W1[:DE])
    m128 = jnp.kron(eye8, jnp.full((DE, DE), 1.0 / DE, jnp.float32))
    b1t = jnp.tile(b1, 8).reshape(1, DN)
    g1t = jnp.tile(g1, 8).reshape(1, DN)
    be1t = jnp.tile(be1, 8).reshape(1, DN)

    newe_p = _eln(edge_features.reshape(EP, DN),
                  gsrc.reshape(EP, DN), gtgt.reshape(EP, DN),
                  a128, m128, b1t, g1t, be1t)
    new_edges = newe_p.reshape(E, DE)

    partials = _sc_scatter(new_edges, src3, tgt3)

    new_nodes = _final(node_features, partials[0, :N], partials[1, :N],
                       W2[:DN], W2[DN:], b2.reshape(1, DN),
                       g2.reshape(1, DN), be2.reshape(1, DN))
    return (new_nodes, new_edges)


# speedup vs baseline: 10.4918x; 1.0124x over previous
import functools

import jax
import jax.numpy as jnp
from jax import lax
from jax.experimental import pallas as pl
from jax.experimental.pallas import tpu as pltpu
from jax.experimental.pallas import tpu_sc as plsc

N = 10000
E = 320000
DN = 128
DE = 16
EP = E // 8

NC = 2
NS = 16
NW = NC * NS
EPW = E // NW
SUB = 100
CH = 2000
NSUB = CH // SUB
NCH = EPW // CH
SPW = EPW // SUB
NP = 10240
RPS = NP // NS
CROWS = CH * DE // DN


def _proj_body(nf_ref, wn_ref, ps_ref, pt_ref):
    p = jnp.dot(nf_ref[...], wn_ref[...], preferred_element_type=jnp.float32)
    ps_ref[...] = p[:, :DE]
    pt_ref[...] = p[:, DE:]


def _proj(node_features, wn):
    bn = 1000
    return pl.pallas_call(
        _proj_body,
        grid=(N // bn,),
        in_specs=[
            pl.BlockSpec((bn, DN), lambda i: (i, 0)),
            pl.BlockSpec((DN, 2 * DE), lambda i: (0, 0)),
        ],
        out_specs=[
            pl.BlockSpec((bn, DE), lambda i: (i, 0)),
            pl.BlockSpec((bn, DE), lambda i: (i, 0)),
        ],
        out_shape=[
            jax.ShapeDtypeStruct((N, DE), jnp.float32),
            jax.ShapeDtypeStruct((N, DE), jnp.float32),
        ],
    )(node_features, wn)


_sc_mesh = plsc.VectorSubcoreMesh(
    core_axis_name="c", subcore_axis_name="s", num_cores=NC, num_subcores=NS
)


@functools.partial(
    pl.kernel,
    out_type=(
        jax.ShapeDtypeStruct((E, DE), jnp.float32),
        jax.ShapeDtypeStruct((E, DE), jnp.float32),
    ),
    mesh=_sc_mesh,
    scratch_types=[
        pltpu.VMEM((NSUB, SUB), jnp.int32),
        pltpu.VMEM((NSUB, SUB), jnp.int32),
        pltpu.VMEM((CH, DE), jnp.float32),
        pltpu.VMEM((CH, DE), jnp.float32),
        pltpu.SemaphoreType.DMA,
    ],
    compiler_params=pltpu.CompilerParams(use_tc_tiling_on_sc=False),
)
def _sc_gather(src_hbm, tgt_hbm, psrc_hbm, ptgt_hbm,
               gsrc_hbm, gtgt_hbm,
               sidx_v, tidx_v, gsrc_v, gtgt_v, sem):
    c = lax.axis_index("c")
    s = lax.axis_index("s")
    w = s * NC + c

    def chunk_body(k, carry):
        base = w * EPW + k * CH
        irow = w * SPW + k * NSUB
        pltpu.sync_copy(src_hbm.at[pl.ds(irow, NSUB)], sidx_v)
        pltpu.sync_copy(tgt_hbm.at[pl.ds(irow, NSUB)], tidx_v)
        hs = []
        for j in range(NSUB):
            hs.append(pltpu.async_copy(
                psrc_hbm.at[sidx_v.at[j]],
                gsrc_v.at[pl.ds(j * SUB, SUB)], sem))
            hs.append(pltpu.async_copy(
                ptgt_hbm.at[tidx_v.at[j]],
                gtgt_v.at[pl.ds(j * SUB, SUB)], sem))
        for h in hs:
            h.wait()
        pltpu.sync_copy(gsrc_v, gsrc_hbm.at[pl.ds(base, CH)])
        pltpu.sync_copy(gtgt_v, gtgt_hbm.at[pl.ds(base, CH)])
        return carry

    lax.fori_loop(0, NCH, chunk_body, 0)


def _eln_body(x_ref, gs_ref, gt_ref, a_ref, m_ref, b1_ref, g1_ref, be1_ref,
              out_ref):
    x = x_ref[...]
    y = (lax.dot(x, a_ref[...], preferred_element_type=jnp.float32)
         + b1_ref[...] + gs_ref[...] + gt_ref[...])
    mu = lax.dot(y, m_ref[...], preferred_element_type=jnp.float32)
    d = y - mu
    var = lax.dot(d * d, m_ref[...], preferred_element_type=jnp.float32)
    out_ref[...] = d * lax.rsqrt(var + 1e-5) * g1_ref[...] + be1_ref[...]


def _eln(xp, gsp, gtp, a128, m128, b1t, g1t, be1t):
    bp = 2000
    return pl.pallas_call(
        _eln_body,
        grid=(EP // bp,),
        in_specs=[
            pl.BlockSpec((bp, DN), lambda i: (i, 0)),
            pl.BlockSpec((bp, DN), lambda i: (i, 0)),
            pl.BlockSpec((bp, DN), lambda i: (i, 0)),
            pl.BlockSpec((DN, DN), lambda i: (0, 0)),
            pl.BlockSpec((DN, DN), lambda i: (0, 0)),
            pl.BlockSpec((1, DN), lambda i: (0, 0)),
            pl.BlockSpec((1, DN), lambda i: (0, 0)),
            pl.BlockSpec((1, DN), lambda i: (0, 0)),
        ],
        out_specs=pl.BlockSpec((bp, DN), lambda i: (i, 0)),
        out_shape=jax.ShapeDtypeStruct((EP, DN), jnp.float32),
    )(xp, gsp, gtp, a128, m128, b1t, g1t, be1t)


@functools.partial(
    pl.kernel,
    out_type=jax.ShapeDtypeStruct((NC, NP, 2 * DE), jnp.float32),
    mesh=_sc_mesh,
    scratch_types=[
        pltpu.VMEM((NSUB, SUB), jnp.int32),
        pltpu.VMEM((NSUB, SUB), jnp.int32),
        pltpu.VMEM((CH, 2 * DE), jnp.float32),
        pltpu.VMEM((RPS, 2 * DE), jnp.float32),
        pltpu.VMEM_SHARED((NP, 2 * DE), jnp.float32),
        pltpu.SemaphoreType.DMA,
    ],
    compiler_params=pltpu.CompilerParams(use_tc_tiling_on_sc=False),
)
def _sc_scatter(newe_hbm, src_hbm, tgt_hbm, part_hbm,
                sidx_v, tidx_v, vals_v, zbuf_v, acc_sh, sem):
    c = lax.axis_index("c")
    s = lax.axis_index("s")
    w = s * NC + c

    z16 = jnp.zeros((DE,), jnp.float32)
    o16 = jnp.full((DE,), 1.0, jnp.float32)

    def zb_body(i, carry):
        zbuf_v[i, pl.ds(0, DE)] = z16
        zbuf_v[i, pl.ds(DE, DE)] = z16
        return carry

    lax.fori_loop(0, RPS, zb_body, 0)
    pltpu.sync_copy(zbuf_v, acc_sh.at[pl.ds(s * RPS, RPS)])

    def ones_body(i, carry):
        vals_v[i, pl.ds(DE, DE)] = o16
        return carry

    lax.fori_loop(0, CH, ones_body, 0)

    plsc.subcore_barrier()

    def chunk_body(k, carry):
        base = w * EPW + k * CH
        irow = w * SPW + k * NSUB
        pltpu.sync_copy(src_hbm.at[pl.ds(irow, NSUB)], sidx_v)
        pltpu.sync_copy(tgt_hbm.at[pl.ds(irow, NSUB)], tidx_v)
        pltpu.sync_copy(newe_hbm.at[pl.ds(base, CH)],
                        vals_v.at[:, pl.ds(0, DE)])
        for j in range(NSUB):
            vrow = vals_v.at[pl.ds(j * SUB, SUB)]
            pltpu.sync_copy(vrow, acc_sh.at[sidx_v.at[j]], add=True)
            pltpu.sync_copy(vrow, acc_sh.at[tidx_v.at[j]], add=True)
        return carry

    lax.fori_loop(0, NCH, chunk_body, 0)

    plsc.subcore_barrier()
    pltpu.sync_copy(acc_sh.at[pl.ds(s * RPS, RPS)],
                    part_hbm.at[c, pl.ds(s * RPS, RPS)])


def _final_body(nf_ref, p0_ref, p1_ref, w2a_ref, w2b_ref, b2_ref, g2_ref,
                be2_ref, out_ref):
    sums = p0_ref[...] + p1_ref[...]
    emean = sums[:, :DE] / (sums[:, DE:DE + 1] + 1e-10)
    nf = nf_ref[...]
    h = (nf
         + jnp.dot(nf, w2a_ref[...], preferred_element_type=jnp.float32)
         + jnp.dot(emean, w2b_ref[...], preferred_element_type=jnp.float32)
         + b2_ref[...])
    mu = jnp.mean(h, axis=-1, keepdims=True)
    d = h - mu
    var = jnp.mean(d * d, axis=-1, keepdims=True)
    out_ref[...] = d * lax.rsqrt(var + 1e-5) * g2_ref[...] + be2_ref[...]


def _final(node_features, p0, p1, w2a, w2b, b2row, g2row, be2row):
    bn = 400
    return pl.pallas_call(
        _final_body,
        grid=(N // bn,),
        in_specs=[
            pl.BlockSpec((bn, DN), lambda i: (i, 0)),
            pl.BlockSpec((bn, 2 * DE), lambda i: (i, 0)),
            pl.BlockSpec((bn, 2 * DE), lambda i: (i, 0)),
            pl.BlockSpec((DN, DN), lambda i: (0, 0)),
            pl.BlockSpec((DE, DN), lambda i: (0, 0)),
            pl.BlockSpec((1, DN), lambda i: (0, 0)),
            pl.BlockSpec((1, DN), lambda i: (0, 0)),
            pl.BlockSpec((1, DN), lambda i: (0, 0)),
        ],
        out_specs=pl.BlockSpec((bn, DN), lambda i: (i, 0)),
        out_shape=jax.ShapeDtypeStruct((N, DN), jnp.float32),
    )(node_features, p0, p1, w2a, w2b, b2row, g2row, be2row)


def kernel(node_features, edge_features, edge_index, W1, b1, g1, be1,
           W2, b2, g2, be2):
    src3 = edge_index[0].reshape(E // SUB, SUB)
    tgt3 = edge_index[1].reshape(E // SUB, SUB)

    wn = jnp.concatenate([W1[DE:DE + DN], W1[DE + DN:]], axis=1)
    psrc, ptgt = _proj(node_features, wn)

    gsrc, gtgt = _sc_gather(src3, tgt3, psrc, ptgt)

    eye8 = jnp.eye(8, dtype=jnp.float32)
    a128 = jnp.eye(DN, dtype=jnp.float32) + jnp.kron(eye8, W1[:DE])
    m128 = jnp.kron(eye8, jnp.full((DE, DE), 1.0 / DE, jnp.float32))
    b1t = jnp.tile(b1, 8).reshape(1, DN)
    g1t = jnp.tile(g1, 8).reshape(1, DN)
    be1t = jnp.tile(be1, 8).reshape(1, DN)

    newe_p = _eln(edge_features.reshape(EP, DN),
                  gsrc.reshape(EP, DN), gtgt.reshape(EP, DN),
                  a128, m128, b1t, g1t, be1t)
    new_edges = newe_p.reshape(E, DE)

    partials = _sc_scatter(new_edges, src3, tgt3)

    new_nodes = _final(node_features, partials[0, :N], partials[1, :N],
                       W2[:DN], W2[DN:], b2.reshape(1, DN),
                       g2.reshape(1, DN), be2.reshape(1, DN))
    return (new_nodes, new_edges)
